# baseline (speedup 1.0000x reference)
import functools

import jax
import jax.numpy as jnp
import numpy as np
from jax import lax
from jax.experimental import pallas as pl
from jax.experimental.pallas import tpu as pltpu
from jax.experimental.pallas import tpu_sc as plsc


def kernel(in_features, neighbors_index, neighbors_row_splits):
    N, C = in_features.shape
    E = neighbors_index.shape[0]
    M = neighbors_row_splits.shape[0] - 1

    info = plsc.get_sparse_core_info()
    NCORES, NSUB = info.num_cores, info.num_subcores
    NW = NCORES * NSUB
    MPW = -(-((M + NW - 1) // NW) // 8) * 8
    MLAST = M - (NW - 1) * MPW
    G = 32
    R = 8
    W = 4
    BLK = 2048
    LAN = 16
    KC = C // LAN
    CW = C // 2
    KW = CW // LAN

    tbl32 = jax.lax.bitcast_convert_type(
        in_features.astype(jnp.bfloat16).reshape(N, CW, 2), jnp.int32)

    RSBUF = ((MPW + 2 + 7 + 15 + 7) // 8) * 8
    idx32 = jnp.pad(neighbors_index.astype(jnp.int32), (0, BLK + G + 8))
    rs32 = jnp.pad(neighbors_row_splits.astype(jnp.int32), (0, RSBUF),
                   constant_values=E)

    mesh = plsc.VectorSubcoreMesh(core_axis_name="c", subcore_axis_name="s")

    @functools.partial(
        pl.kernel,
        mesh=mesh,
        out_type=jax.ShapeDtypeStruct((M, C), jnp.float32),
        scratch_types=[
            pltpu.VMEM((RSBUF,), jnp.int32),
            pltpu.VMEM((BLK,), jnp.int32),
            pltpu.VMEM((R, G, CW), jnp.int32),
            pltpu.VMEM((W * C,), jnp.float32),
            pltpu.SemaphoreType.DMA((R,)),
            pltpu.SemaphoreType.DMA((W,)),
        ],
    )
    def pool(feat_hbm, idx_hbm, rs_hbm, out_hbm,
             rs_v, blk_v, g_v, row_v, gsem, wsem):
        wid = lax.axis_index("s") * NCORES + lax.axis_index("c")
        m0 = pl.multiple_of(wid * MPW, 8)
        mcount = jnp.minimum(MPW, M - m0)
        rs_astart = pl.multiple_of((m0 // 8) * 8, 8)
        rs_off = m0 - rs_astart
        pltpu.sync_copy(rs_hbm.at[pl.ds(rs_astart, RSBUF)], rs_v)

        def rs_at(i):
            return rs_v[pl.ds(rs_off + i, LAN)][0]

        e0 = rs_at(0)
        eN = rs_at(mcount)
        gstart = pl.multiple_of((e0 // 8) * 8, 8)
        nchunks = (eN - gstart + G - 1) // G
        nevents = nchunks + mcount

        zero = jnp.zeros((LAN,), jnp.float32)
        zeros_kc = (zero,) * KC

        def issue_gather(off, slot_ds):
            for h in range(G // LAN):
                idxv = blk_v[pl.ds(off + h * LAN, LAN)]
                pltpu.async_copy(feat_hbm.at[idxv],
                                 g_v.at[slot_ds, pl.ds(h * LAN, LAN)],
                                 gsem.at[slot_ds])

        @pl.when(nchunks > 0)
        def _():
            pltpu.sync_copy(idx_hbm.at[pl.ds(gstart, BLK)], blk_v)
            for i in range(R - 1):
                @pl.when(nchunks > i)
                def _(i=i):
                    issue_gather(i * G, i)
            pltpu.make_async_copy(feat_hbm.at[pl.ds(0, G)], g_v.at[0],
                                  gsem.at[0]).wait()

        def event(_, st):
            c, cur, m, seg_start, seg_end, bstart, accs = st
            p = lax.rem(c, R)
            cs = gstart + c * G
            cend = jnp.minimum(cs + G, eN)

            take = jnp.maximum(jnp.minimum(seg_end, cend) - cur, 0)
            lo = cur - cs

            mask_hi = jnp.full((LAN,), -65536, jnp.int32)

            def edge_body(e, a):
                out = list(a)
                for k in range(KW):
                    w = g_v[p, e, pl.ds(k * LAN, LAN)]
                    lo = lax.bitcast_convert_type(w << 16, jnp.float32)
                    hi = lax.bitcast_convert_type(w & mask_hi, jnp.float32)
                    out[2 * k] = out[2 * k] + lo
                    out[2 * k + 1] = out[2 * k + 1] + hi
                return tuple(out)

            accs = lax.fori_loop(lo, lo + take, edge_body, accs)
            cur = cur + take

            hit = jnp.logical_and(cur >= seg_end, m < mcount)
            adv = jnp.logical_and(jnp.logical_not(hit),
                                  jnp.logical_and(cur >= cend,
                                                  c + 1 < nchunks))

            @pl.when(hit)
            def _():
                ws2 = lax.rem(m, W)
                rbase = pl.multiple_of(ws2 * C, C)
                rslice = row_v.at[pl.ds(rbase, C)]

                @pl.when(m >= W)
                def _():
                    pltpu.make_async_copy(out_hbm.at[m0], rslice,
                                          wsem.at[ws2]).wait()

                cnt = seg_end - seg_start
                cntv = jnp.full((LAN,), cnt.astype(jnp.float32))
                recip = 1.0 / jnp.maximum(cntv, 1.0)
                for k in range(KC):
                    row_v[pl.ds(rbase + k * LAN, LAN)] = accs[k] * recip
                pltpu.async_copy(rslice, out_hbm.at[m0 + m], wsem.at[ws2])

            issue = jnp.logical_and(adv, c + R - 1 < nchunks)
            naddr = pl.multiple_of(gstart + (c + R - 1) * G, 8)
            restage = jnp.logical_and(issue, naddr + G > bstart + BLK)
            nbstart = jnp.where(restage, naddr, bstart)

            @pl.when(adv)
            def _():
                @pl.when(issue)
                def _():
                    @pl.when(restage)
                    def _():
                        pltpu.sync_copy(idx_hbm.at[pl.ds(naddr, BLK)], blk_v)

                    issue_gather(naddr - nbstart, lax.rem(c + R - 1, R))

                ws = lax.rem(c + 1, R)
                pltpu.make_async_copy(feat_hbm.at[pl.ds(0, G)],
                                      g_v.at[ws], gsem.at[ws]).wait()

            nm = jnp.where(hit, m + 1, m)
            seg_start = jnp.where(hit, seg_end, seg_start)
            seg_end = jnp.where(hit, rs_at(nm + 1), seg_end)
            accs = tuple(jnp.where(hit, zero, a) for a in accs)
            c = jnp.where(adv, c + 1, c)
            return (c, cur, nm, seg_start, seg_end, nbstart, accs)

        st0 = (jnp.int32(0), e0, jnp.int32(0), e0, rs_at(1),
               gstart, zeros_kc)
        lax.fori_loop(0, nevents, event, st0)

        for i in range(W):
            @pl.when(mcount > i)
            def _(i=i):
                ds = lax.rem(mcount - 1 - i, W)
                dbase = pl.multiple_of(ds * C, C)
                pltpu.make_async_copy(out_hbm.at[m0],
                                      row_v.at[pl.ds(dbase, C)],
                                      wsem.at[ds]).wait()

    out_raw = pool(tbl32, idx32, rs32)
    return (out_raw.reshape(M, KW, 2, LAN).transpose(0, 1, 3, 2)
            .reshape(M, C))


# speedup vs baseline: 2.1558x; 2.1558x over previous
import functools

import jax
import jax.numpy as jnp
from jax import lax
from jax.experimental import pallas as pl
from jax.experimental.pallas import tpu as pltpu
from jax.experimental.pallas import tpu_sc as plsc


def kernel(in_features, neighbors_index, neighbors_row_splits):
    N, C = in_features.shape
    E = neighbors_index.shape[0]
    M = neighbors_row_splits.shape[0] - 1

    info = plsc.get_sparse_core_info()
    NCORES, NSUB = info.num_cores, info.num_subcores
    NW = NCORES * NSUB
    MPW = -(-((M + NW - 1) // NW) // 8) * 8
    G = 32
    R = 8
    W = 4
    BLK = 2048
    LAN = 16
    KC = C // LAN
    CW = C // 2
    KW = CW // LAN
    NPW = -(-((N + NW - 1) // NW) // 8) * 8
    RB = 32

    RSBUF = ((MPW + 2 + 7 + 15 + 7) // 8) * 8
    idx32 = jnp.pad(neighbors_index.astype(jnp.int32), (0, BLK + G + 8))
    rs32 = jnp.pad(neighbors_row_splits.astype(jnp.int32), (0, RSBUF),
                   constant_values=E)

    mesh = plsc.VectorSubcoreMesh(core_axis_name="c", subcore_axis_name="s")

    @functools.partial(
        pl.kernel,
        mesh=mesh,
        out_type=jax.ShapeDtypeStruct((N, CW), jnp.int32),
        scratch_types=[
            pltpu.VMEM((2, RB, C), jnp.float32),
            pltpu.VMEM((RB, CW), jnp.int32),
            pltpu.SemaphoreType.DMA((2,)),
        ],
    )
    def pack(feat_hbm, tbl_hbm, fin_v, pout_v, rsem):
        wid = lax.axis_index("s") * NCORES + lax.axis_index("c")
        n0 = pl.multiple_of(wid * NPW, 8)
        nrows = jnp.minimum(NPW, N - n0)
        nch = (nrows + RB - 1) // RB

        def row0_of(ch):
            return pl.multiple_of(
                jnp.minimum(n0 + ch * RB, N - RB), 8)

        @pl.when(nch > 0)
        def _():
            pltpu.async_copy(feat_hbm.at[pl.ds(row0_of(0), RB)],
                             fin_v.at[0], rsem.at[0])

        def chunk(ch, _):
            slot = lax.rem(ch, 2)

            @pl.when(ch + 1 < nch)
            def _():
                pltpu.async_copy(feat_hbm.at[pl.ds(row0_of(ch + 1), RB)],
                                 fin_v.at[1 - slot], rsem.at[1 - slot])

            pltpu.make_async_copy(feat_hbm.at[pl.ds(0, RB)], fin_v.at[slot],
                                  rsem.at[slot]).wait()

            half = jnp.full((LAN,), 0x8000, jnp.int32)
            mask_lo = jnp.full((LAN,), 0xFFFF, jnp.int32)
            mask_hi = jnp.full((LAN,), -65536, jnp.int32)

            def row_body(r, carry):
                for k in range(KW):
                    x0 = lax.bitcast_convert_type(
                        fin_v[slot, r, pl.ds(32 * k, LAN)], jnp.int32)
                    x1 = lax.bitcast_convert_type(
                        fin_v[slot, r, pl.ds(32 * k + LAN, LAN)], jnp.int32)
                    w = (((x0 + half) >> 16) & mask_lo) | \
                        ((x1 + half) & mask_hi)
                    pout_v[r, pl.ds(k * LAN, LAN)] = w
                return carry

            lax.fori_loop(0, RB, row_body, 0)
            pltpu.sync_copy(pout_v, tbl_hbm.at[pl.ds(row0_of(ch), RB)])
            return 0

        lax.fori_loop(0, nch, chunk, 0)

    @functools.partial(
        pl.kernel,
        mesh=mesh,
        out_type=jax.ShapeDtypeStruct((M, C), jnp.float32),
        scratch_types=[
            pltpu.VMEM((RSBUF,), jnp.int32),
            pltpu.VMEM((BLK,), jnp.int32),
            pltpu.VMEM((R, G, CW), jnp.int32),
            pltpu.VMEM((W * C,), jnp.float32),
            pltpu.SemaphoreType.DMA((R,)),
            pltpu.SemaphoreType.DMA((W,)),
        ],
    )
    def pool(feat_hbm, idx_hbm, rs_hbm, out_hbm,
             rs_v, blk_v, g_v, row_v, gsem, wsem):
        wid = lax.axis_index("s") * NCORES + lax.axis_index("c")
        m0 = pl.multiple_of(wid * MPW, 8)
        mcount = jnp.minimum(MPW, M - m0)
        rs_astart = pl.multiple_of((m0 // 8) * 8, 8)
        rs_off = m0 - rs_astart
        pltpu.sync_copy(rs_hbm.at[pl.ds(rs_astart, RSBUF)], rs_v)

        def rs_at(i):
            return rs_v[pl.ds(rs_off + i, LAN)][0]

        e0 = rs_at(0)
        eN = rs_at(mcount)
        gstart = pl.multiple_of((e0 // 8) * 8, 8)
        nchunks = (eN - gstart + G - 1) // G
        nevents = nchunks + mcount

        zero = jnp.zeros((LAN,), jnp.float32)
        zeros_kc = (zero,) * KC

        def issue_gather(off, slot_ds):
            for h in range(G // LAN):
                idxv = blk_v[pl.ds(off + h * LAN, LAN)]
                pltpu.async_copy(feat_hbm.at[idxv],
                                 g_v.at[slot_ds, pl.ds(h * LAN, LAN)],
                                 gsem.at[slot_ds])

        @pl.when(nchunks > 0)
        def _():
            pltpu.sync_copy(idx_hbm.at[pl.ds(gstart, BLK)], blk_v)
            for i in range(R - 1):
                @pl.when(nchunks > i)
                def _(i=i):
                    issue_gather(i * G, i)
            pltpu.make_async_copy(feat_hbm.at[pl.ds(0, G)], g_v.at[0],
                                  gsem.at[0]).wait()

        def event(_, st):
            c, cur, m, seg_start, seg_end, bstart, accs = st
            p = lax.rem(c, R)
            cs = gstart + c * G
            cend = jnp.minimum(cs + G, eN)

            take = jnp.maximum(jnp.minimum(seg_end, cend) - cur, 0)
            lo = cur - cs

            mask_hi = jnp.full((LAN,), -65536, jnp.int32)

            def edge_body(e, a):
                out = list(a)
                for k in range(KW):
                    w = g_v[p, e, pl.ds(k * LAN, LAN)]
                    lo16 = lax.bitcast_convert_type(w << 16, jnp.float32)
                    hi16 = lax.bitcast_convert_type(w & mask_hi, jnp.float32)
                    out[2 * k] = out[2 * k] + lo16
                    out[2 * k + 1] = out[2 * k + 1] + hi16
                return tuple(out)

            accs = lax.fori_loop(lo, lo + take, edge_body, accs)
            cur = cur + take

            hit = jnp.logical_and(cur >= seg_end, m < mcount)
            adv = jnp.logical_and(jnp.logical_not(hit),
                                  jnp.logical_and(cur >= cend,
                                                  c + 1 < nchunks))

            @pl.when(hit)
            def _():
                ws2 = lax.rem(m, W)
                rbase = pl.multiple_of(ws2 * C, C)
                rslice = row_v.at[pl.ds(rbase, C)]

                @pl.when(m >= W)
                def _():
                    pltpu.make_async_copy(out_hbm.at[m0], rslice,
                                          wsem.at[ws2]).wait()

                cnt = seg_end - seg_start
                cntv = jnp.full((LAN,), cnt.astype(jnp.float32))
                recip = 1.0 / jnp.maximum(cntv, 1.0)
                for a in range(KC):
                    row_v[pl.ds(rbase + a * LAN, LAN)] = accs[a] * recip
                pltpu.async_copy(rslice, out_hbm.at[m0 + m], wsem.at[ws2])

            issue = jnp.logical_and(adv, c + R - 1 < nchunks)
            naddr = pl.multiple_of(gstart + (c + R - 1) * G, 8)
            restage = jnp.logical_and(issue, naddr + G > bstart + BLK)
            nbstart = jnp.where(restage, naddr, bstart)

            @pl.when(adv)
            def _():
                @pl.when(issue)
                def _():
                    @pl.when(restage)
                    def _():
                        pltpu.sync_copy(idx_hbm.at[pl.ds(naddr, BLK)], blk_v)

                    issue_gather(naddr - nbstart, lax.rem(c + R - 1, R))

                ws = lax.rem(c + 1, R)
                pltpu.make_async_copy(feat_hbm.at[pl.ds(0, G)],
                                      g_v.at[ws], gsem.at[ws]).wait()

            nm = jnp.where(hit, m + 1, m)
            seg_start = jnp.where(hit, seg_end, seg_start)
            seg_end = jnp.where(hit, rs_at(nm + 1), seg_end)
            accs = tuple(jnp.where(hit, zero, a) for a in accs)
            c = jnp.where(adv, c + 1, c)
            return (c, cur, nm, seg_start, seg_end, nbstart, accs)

        st0 = (jnp.int32(0), e0, jnp.int32(0), e0, rs_at(1),
               gstart, zeros_kc)
        lax.fori_loop(0, nevents, event, st0)

        for i in range(W):
            @pl.when(mcount > i)
            def _(i=i):
                ds = lax.rem(mcount - 1 - i, W)
                dbase = pl.multiple_of(ds * C, C)
                pltpu.make_async_copy(out_hbm.at[m0],
                                      row_v.at[pl.ds(dbase, C)],
                                      wsem.at[ds]).wait()

    return pool(pack(in_features), idx32, rs32)


# speedup vs baseline: 2.1869x; 1.0144x over previous
import functools

import jax
import jax.numpy as jnp
from jax import lax
from jax.experimental import pallas as pl
from jax.experimental.pallas import tpu as pltpu
from jax.experimental.pallas import tpu_sc as plsc


def kernel(in_features, neighbors_index, neighbors_row_splits):
    N, C = in_features.shape
    E = neighbors_index.shape[0]
    M = neighbors_row_splits.shape[0] - 1

    info = plsc.get_sparse_core_info()
    NCORES, NSUB = info.num_cores, info.num_subcores
    NW = NCORES * NSUB
    MPW = -(-((M + NW - 1) // NW) // 8) * 8
    G = 32
    R = 8
    W = 4
    BLK = 2048
    LAN = 16
    KC = C // LAN
    CW = C // 2
    KW = CW // LAN
    NPW = -(-((N + NW - 1) // NW) // 8) * 8
    RB = 32

    RSBUF = ((MPW + 2 + 7 + 15 + 7) // 8) * 8
    idx32 = jnp.pad(neighbors_index.astype(jnp.int32), (0, BLK + G + 8))
    rs32 = jnp.pad(neighbors_row_splits.astype(jnp.int32), (0, RSBUF),
                   constant_values=E)

    mesh = plsc.VectorSubcoreMesh(core_axis_name="c", subcore_axis_name="s")

    @functools.partial(
        pl.kernel,
        mesh=mesh,
        out_type=jax.ShapeDtypeStruct((N, CW), jnp.int32),
        scratch_types=[
            pltpu.VMEM((2, RB, C), jnp.float32),
            pltpu.VMEM((2, RB, CW), jnp.int32),
            pltpu.SemaphoreType.DMA((2,)),
            pltpu.SemaphoreType.DMA((2,)),
        ],
    )
    def pack(feat_hbm, tbl_hbm, fin_v, pout_v, rsem, wsem):
        wid = lax.axis_index("s") * NCORES + lax.axis_index("c")
        n0 = pl.multiple_of(wid * NPW, 8)
        nrows = jnp.minimum(NPW, N - n0)
        nch = (nrows + RB - 1) // RB

        def row0_of(ch):
            return pl.multiple_of(
                jnp.minimum(n0 + ch * RB, N - RB), 8)

        @pl.when(nch > 0)
        def _():
            pltpu.async_copy(feat_hbm.at[pl.ds(row0_of(0), RB)],
                             fin_v.at[0], rsem.at[0])

        def chunk(ch, _):
            slot = lax.rem(ch, 2)

            @pl.when(ch + 1 < nch)
            def _():
                pltpu.async_copy(feat_hbm.at[pl.ds(row0_of(ch + 1), RB)],
                                 fin_v.at[1 - slot], rsem.at[1 - slot])

            pltpu.make_async_copy(feat_hbm.at[pl.ds(0, RB)], fin_v.at[slot],
                                  rsem.at[slot]).wait()

            half = jnp.full((LAN,), 0x8000, jnp.int32)
            mask_lo = jnp.full((LAN,), 0xFFFF, jnp.int32)
            mask_hi = jnp.full((LAN,), -65536, jnp.int32)

            @pl.when(ch >= 2)
            def _():
                pltpu.make_async_copy(tbl_hbm.at[pl.ds(0, RB)],
                                      pout_v.at[slot], wsem.at[slot]).wait()

            def row_body(r, carry):
                for k in range(KW):
                    x0 = lax.bitcast_convert_type(
                        fin_v[slot, r, pl.ds(32 * k, LAN)], jnp.int32)
                    x1 = lax.bitcast_convert_type(
                        fin_v[slot, r, pl.ds(32 * k + LAN, LAN)], jnp.int32)
                    w = (((x0 + half) >> 16) & mask_lo) | \
                        ((x1 + half) & mask_hi)
                    pout_v[slot, r, pl.ds(k * LAN, LAN)] = w
                return carry

            lax.fori_loop(0, RB, row_body, 0)
            pltpu.async_copy(pout_v.at[slot], tbl_hbm.at[pl.ds(row0_of(ch), RB)],
                             wsem.at[slot])
            return 0

        lax.fori_loop(0, nch, chunk, 0)

        for i in range(2):
            @pl.when(nch > i)
            def _(i=i):
                ds = lax.rem(nch - 1 - i, 2)
                pltpu.make_async_copy(tbl_hbm.at[pl.ds(0, RB)],
                                      pout_v.at[ds], wsem.at[ds]).wait()

    @functools.partial(
        pl.kernel,
        mesh=mesh,
        out_type=jax.ShapeDtypeStruct((M, C), jnp.float32),
        scratch_types=[
            pltpu.VMEM((RSBUF,), jnp.int32),
            pltpu.VMEM((BLK,), jnp.int32),
            pltpu.VMEM((R, G, CW), jnp.int32),
            pltpu.VMEM((W * C,), jnp.float32),
            pltpu.SemaphoreType.DMA((R,)),
            pltpu.SemaphoreType.DMA((W,)),
        ],
    )
    def pool(feat_hbm, idx_hbm, rs_hbm, out_hbm,
             rs_v, blk_v, g_v, row_v, gsem, wsem):
        wid = lax.axis_index("s") * NCORES + lax.axis_index("c")
        m0 = pl.multiple_of(wid * MPW, 8)
        mcount = jnp.minimum(MPW, M - m0)
        rs_astart = pl.multiple_of((m0 // 8) * 8, 8)
        rs_off = m0 - rs_astart
        pltpu.sync_copy(rs_hbm.at[pl.ds(rs_astart, RSBUF)], rs_v)

        def rs_at(i):
            return rs_v[pl.ds(rs_off + i, LAN)][0]

        e0 = rs_at(0)
        eN = rs_at(mcount)
        gstart = pl.multiple_of((e0 // 8) * 8, 8)
        nchunks = (eN - gstart + G - 1) // G
        nevents = nchunks + mcount

        zero = jnp.zeros((LAN,), jnp.float32)
        zeros_kc = (zero,) * KC

        def issue_gather(off, slot_ds):
            for h in range(G // LAN):
                idxv = blk_v[pl.ds(off + h * LAN, LAN)]
                pltpu.async_copy(feat_hbm.at[idxv],
                                 g_v.at[slot_ds, pl.ds(h * LAN, LAN)],
                                 gsem.at[slot_ds])

        @pl.when(nchunks > 0)
        def _():
            pltpu.sync_copy(idx_hbm.at[pl.ds(gstart, BLK)], blk_v)
            for i in range(R - 1):
                @pl.when(nchunks > i)
                def _(i=i):
                    issue_gather(i * G, i)
            pltpu.make_async_copy(feat_hbm.at[pl.ds(0, G)], g_v.at[0],
                                  gsem.at[0]).wait()

        def event(_, st):
            c, cur, m, seg_start, seg_end, bstart, accs = st
            p = lax.rem(c, R)
            cs = gstart + c * G
            cend = jnp.minimum(cs + G, eN)

            take = jnp.maximum(jnp.minimum(seg_end, cend) - cur, 0)
            lo = cur - cs

            mask_hi = jnp.full((LAN,), -65536, jnp.int32)

            def edge_body(e, a):
                out = list(a)
                for k in range(KW):
                    w = g_v[p, e, pl.ds(k * LAN, LAN)]
                    lo16 = lax.bitcast_convert_type(w << 16, jnp.float32)
                    hi16 = lax.bitcast_convert_type(w & mask_hi, jnp.float32)
                    out[2 * k] = out[2 * k] + lo16
                    out[2 * k + 1] = out[2 * k + 1] + hi16
                return tuple(out)

            accs = lax.fori_loop(lo, lo + take, edge_body, accs)
            cur = cur + take

            hit = jnp.logical_and(cur >= seg_end, m < mcount)
            adv = jnp.logical_and(jnp.logical_not(hit),
                                  jnp.logical_and(cur >= cend,
                                                  c + 1 < nchunks))

            @pl.when(hit)
            def _():
                ws2 = lax.rem(m, W)
                rbase = pl.multiple_of(ws2 * C, C)
                rslice = row_v.at[pl.ds(rbase, C)]

                @pl.when(m >= W)
                def _():
                    pltpu.make_async_copy(out_hbm.at[m0], rslice,
                                          wsem.at[ws2]).wait()

                cnt = seg_end - seg_start
                cntv = jnp.full((LAN,), cnt.astype(jnp.float32))
                recip = 1.0 / jnp.maximum(cntv, 1.0)
                for a in range(KC):
                    row_v[pl.ds(rbase + a * LAN, LAN)] = accs[a] * recip
                pltpu.async_copy(rslice, out_hbm.at[m0 + m], wsem.at[ws2])

            issue = jnp.logical_and(adv, c + R - 1 < nchunks)
            naddr = pl.multiple_of(gstart + (c + R - 1) * G, 8)
            restage = jnp.logical_and(issue, naddr + G > bstart + BLK)
            nbstart = jnp.where(restage, naddr, bstart)

            @pl.when(adv)
            def _():
                @pl.when(issue)
                def _():
                    @pl.when(restage)
                    def _():
                        pltpu.sync_copy(idx_hbm.at[pl.ds(naddr, BLK)], blk_v)

                    issue_gather(naddr - nbstart, lax.rem(c + R - 1, R))

                ws = lax.rem(c + 1, R)
                pltpu.make_async_copy(feat_hbm.at[pl.ds(0, G)],
                                      g_v.at[ws], gsem.at[ws]).wait()

            nm = jnp.where(hit, m + 1, m)
            seg_start = jnp.where(hit, seg_end, seg_start)
            seg_end = jnp.where(hit, rs_at(nm + 1), seg_end)
            accs = tuple(jnp.where(hit, zero, a) for a in accs)
            c = jnp.where(adv, c + 1, c)
            return (c, cur, nm, seg_start, seg_end, nbstart, accs)

        st0 = (jnp.int32(0), e0, jnp.int32(0), e0, rs_at(1),
               gstart, zeros_kc)
        lax.fori_loop(0, nevents, event, st0)

        for i in range(W):
            @pl.when(mcount > i)
            def _(i=i):
                ds = lax.rem(mcount - 1 - i, W)
                dbase = pl.multiple_of(ds * C, C)
                pltpu.make_async_copy(out_hbm.at[m0],
                                      row_v.at[pl.ds(dbase, C)],
                                      wsem.at[ds]).wait()

    return pool(pack(in_features), idx32, rs32)


# speedup vs baseline: 2.2234x; 1.0167x over previous
import functools

import jax
import jax.numpy as jnp
from jax import lax
from jax.experimental import pallas as pl
from jax.experimental.pallas import tpu as pltpu
from jax.experimental.pallas import tpu_sc as plsc


def kernel(in_features, neighbors_index, neighbors_row_splits):
    N, C = in_features.shape
    E = neighbors_index.shape[0]
    M = neighbors_row_splits.shape[0] - 1

    info = plsc.get_sparse_core_info()
    NCORES, NSUB = info.num_cores, info.num_subcores
    NW = NCORES * NSUB
    MPW = -(-((M + NW - 1) // NW) // 8) * 8
    G = 64
    R = 8
    W = 4
    BLK = 2048
    LAN = 16
    KC = C // LAN
    CW = C // 2
    KW = CW // LAN
    NPW = -(-((N + NW - 1) // NW) // 8) * 8
    RB = 32

    RSBUF = ((MPW + 2 + 7 + 15 + 7) // 8) * 8
    idx32 = jnp.pad(neighbors_index.astype(jnp.int32), (0, BLK + G + 8))
    rs32 = jnp.pad(neighbors_row_splits.astype(jnp.int32), (0, RSBUF),
                   constant_values=E)

    mesh = plsc.VectorSubcoreMesh(core_axis_name="c", subcore_axis_name="s")

    @functools.partial(
        pl.kernel,
        mesh=mesh,
        out_type=jax.ShapeDtypeStruct((N, CW), jnp.int32),
        scratch_types=[
            pltpu.VMEM((2, RB, C), jnp.float32),
            pltpu.VMEM((2, RB, CW), jnp.int32),
            pltpu.SemaphoreType.DMA((2,)),
            pltpu.SemaphoreType.DMA((2,)),
        ],
    )
    def pack(feat_hbm, tbl_hbm, fin_v, pout_v, rsem, wsem):
        wid = lax.axis_index("s") * NCORES + lax.axis_index("c")
        n0 = pl.multiple_of(wid * NPW, 8)
        nrows = jnp.minimum(NPW, N - n0)
        nch = (nrows + RB - 1) // RB

        def row0_of(ch):
            return pl.multiple_of(
                jnp.minimum(n0 + ch * RB, N - RB), 8)

        @pl.when(nch > 0)
        def _():
            pltpu.async_copy(feat_hbm.at[pl.ds(row0_of(0), RB)],
                             fin_v.at[0], rsem.at[0])

        def chunk(ch, _):
            slot = lax.rem(ch, 2)

            @pl.when(ch + 1 < nch)
            def _():
                pltpu.async_copy(feat_hbm.at[pl.ds(row0_of(ch + 1), RB)],
                                 fin_v.at[1 - slot], rsem.at[1 - slot])

            pltpu.make_async_copy(feat_hbm.at[pl.ds(0, RB)], fin_v.at[slot],
                                  rsem.at[slot]).wait()

            half = jnp.full((LAN,), 0x8000, jnp.int32)
            mask_lo = jnp.full((LAN,), 0xFFFF, jnp.int32)
            mask_hi = jnp.full((LAN,), -65536, jnp.int32)

            @pl.when(ch >= 2)
            def _():
                pltpu.make_async_copy(tbl_hbm.at[pl.ds(0, RB)],
                                      pout_v.at[slot], wsem.at[slot]).wait()

            def row_body(r, carry):
                for k in range(KW):
                    x0 = lax.bitcast_convert_type(
                        fin_v[slot, r, pl.ds(32 * k, LAN)], jnp.int32)
                    x1 = lax.bitcast_convert_type(
                        fin_v[slot, r, pl.ds(32 * k + LAN, LAN)], jnp.int32)
                    w = (((x0 + half) >> 16) & mask_lo) | \
                        ((x1 + half) & mask_hi)
                    pout_v[slot, r, pl.ds(k * LAN, LAN)] = w
                return carry

            lax.fori_loop(0, RB, row_body, 0)
            pltpu.async_copy(pout_v.at[slot], tbl_hbm.at[pl.ds(row0_of(ch), RB)],
                             wsem.at[slot])
            return 0

        lax.fori_loop(0, nch, chunk, 0)

        for i in range(2):
            @pl.when(nch > i)
            def _(i=i):
                ds = lax.rem(nch - 1 - i, 2)
                pltpu.make_async_copy(tbl_hbm.at[pl.ds(0, RB)],
                                      pout_v.at[ds], wsem.at[ds]).wait()

    @functools.partial(
        pl.kernel,
        mesh=mesh,
        out_type=jax.ShapeDtypeStruct((M, C), jnp.float32),
        scratch_types=[
            pltpu.VMEM((RSBUF,), jnp.int32),
            pltpu.VMEM((BLK,), jnp.int32),
            pltpu.VMEM((R, G, CW), jnp.int32),
            pltpu.VMEM((W * C,), jnp.float32),
            pltpu.SemaphoreType.DMA((R,)),
            pltpu.SemaphoreType.DMA((W,)),
        ],
    )
    def pool(feat_hbm, idx_hbm, rs_hbm, out_hbm,
             rs_v, blk_v, g_v, row_v, gsem, wsem):
        wid = lax.axis_index("s") * NCORES + lax.axis_index("c")
        m0 = pl.multiple_of(wid * MPW, 8)
        mcount = jnp.minimum(MPW, M - m0)
        rs_astart = pl.multiple_of((m0 // 8) * 8, 8)
        rs_off = m0 - rs_astart
        pltpu.sync_copy(rs_hbm.at[pl.ds(rs_astart, RSBUF)], rs_v)

        def rs_at(i):
            return rs_v[pl.ds(rs_off + i, LAN)][0]

        e0 = rs_at(0)
        eN = rs_at(mcount)
        gstart = pl.multiple_of((e0 // 8) * 8, 8)
        nchunks = (eN - gstart + G - 1) // G
        nevents = nchunks + mcount

        zero = jnp.zeros((LAN,), jnp.float32)
        zeros_kc = (zero,) * KC

        def issue_gather(off, slot_ds):
            for h in range(G // LAN):
                idxv = blk_v[pl.ds(off + h * LAN, LAN)]
                pltpu.async_copy(feat_hbm.at[idxv],
                                 g_v.at[slot_ds, pl.ds(h * LAN, LAN)],
                                 gsem.at[slot_ds])

        @pl.when(nchunks > 0)
        def _():
            pltpu.sync_copy(idx_hbm.at[pl.ds(gstart, BLK)], blk_v)
            for i in range(R - 1):
                @pl.when(nchunks > i)
                def _(i=i):
                    issue_gather(i * G, i)
            pltpu.make_async_copy(feat_hbm.at[pl.ds(0, G)], g_v.at[0],
                                  gsem.at[0]).wait()

        def event(_, st):
            c, cur, m, seg_start, seg_end, bstart, accs = st
            p = lax.rem(c, R)
            cs = gstart + c * G
            cend = jnp.minimum(cs + G, eN)

            take = jnp.maximum(jnp.minimum(seg_end, cend) - cur, 0)
            lo = cur - cs

            mask_hi = jnp.full((LAN,), -65536, jnp.int32)

            def edge_body(e, a):
                out = list(a)
                for k in range(KW):
                    w = g_v[p, e, pl.ds(k * LAN, LAN)]
                    lo16 = lax.bitcast_convert_type(w << 16, jnp.float32)
                    hi16 = lax.bitcast_convert_type(w & mask_hi, jnp.float32)
                    out[2 * k] = out[2 * k] + lo16
                    out[2 * k + 1] = out[2 * k + 1] + hi16
                return tuple(out)

            accs = lax.fori_loop(lo, lo + take, edge_body, accs)
            cur = cur + take

            hit = jnp.logical_and(cur >= seg_end, m < mcount)
            adv = jnp.logical_and(jnp.logical_not(hit),
                                  jnp.logical_and(cur >= cend,
                                                  c + 1 < nchunks))

            @pl.when(hit)
            def _():
                ws2 = lax.rem(m, W)
                rbase = pl.multiple_of(ws2 * C, C)
                rslice = row_v.at[pl.ds(rbase, C)]

                @pl.when(m >= W)
                def _():
                    pltpu.make_async_copy(out_hbm.at[m0], rslice,
                                          wsem.at[ws2]).wait()

                cnt = seg_end - seg_start
                cntv = jnp.full((LAN,), cnt.astype(jnp.float32))
                recip = 1.0 / jnp.maximum(cntv, 1.0)
                for a in range(KC):
                    row_v[pl.ds(rbase + a * LAN, LAN)] = accs[a] * recip
                pltpu.async_copy(rslice, out_hbm.at[m0 + m], wsem.at[ws2])

            issue = jnp.logical_and(adv, c + R - 1 < nchunks)
            naddr = pl.multiple_of(gstart + (c + R - 1) * G, 8)
            restage = jnp.logical_and(issue, naddr + G > bstart + BLK)
            nbstart = jnp.where(restage, naddr, bstart)

            @pl.when(adv)
            def _():
                @pl.when(issue)
                def _():
                    @pl.when(restage)
                    def _():
                        pltpu.sync_copy(idx_hbm.at[pl.ds(naddr, BLK)], blk_v)

                    issue_gather(naddr - nbstart, lax.rem(c + R - 1, R))

                ws = lax.rem(c + 1, R)
                pltpu.make_async_copy(feat_hbm.at[pl.ds(0, G)],
                                      g_v.at[ws], gsem.at[ws]).wait()

            nm = jnp.where(hit, m + 1, m)
            seg_start = jnp.where(hit, seg_end, seg_start)
            seg_end = jnp.where(hit, rs_at(nm + 1), seg_end)
            accs = tuple(jnp.where(hit, zero, a) for a in accs)
            c = jnp.where(adv, c + 1, c)
            return (c, cur, nm, seg_start, seg_end, nbstart, accs)

        st0 = (jnp.int32(0), e0, jnp.int32(0), e0, rs_at(1),
               gstart, zeros_kc)
        lax.fori_loop(0, nevents, event, st0)

        for i in range(W):
            @pl.when(mcount > i)
            def _(i=i):
                ds = lax.rem(mcount - 1 - i, W)
                dbase = pl.multiple_of(ds * C, C)
                pltpu.make_async_copy(out_hbm.at[m0],
                                      row_v.at[pl.ds(dbase, C)],
                                      wsem.at[ds]).wait()

    return pool(pack(in_features), idx32, rs32)


# speedup vs baseline: 2.3246x; 1.0455x over previous
import functools

import jax
import jax.numpy as jnp
from jax import lax
from jax.experimental import pallas as pl
from jax.experimental.pallas import tpu as pltpu
from jax.experimental.pallas import tpu_sc as plsc


def kernel(in_features, neighbors_index, neighbors_row_splits):
    N, C = in_features.shape
    E = neighbors_index.shape[0]
    M = neighbors_row_splits.shape[0] - 1

    info = plsc.get_sparse_core_info()
    NCORES, NSUB = info.num_cores, info.num_subcores
    NW = NCORES * NSUB
    MPW = -(-((M + NW - 1) // NW) // 8) * 8
    MLAST = M - (NW - 1) * MPW
    G = 32
    R = 8
    W = 4
    BLK = 2048
    LAN = 16
    KC = C // LAN

    RSBUF = ((MPW + 2 + 7 + 15 + 7) // 8) * 8
    idx32 = jnp.pad(neighbors_index.astype(jnp.int32), (0, BLK + G + 8))
    rs32 = jnp.pad(neighbors_row_splits.astype(jnp.int32), (0, RSBUF),
                   constant_values=E)

    mesh = plsc.VectorSubcoreMesh(core_axis_name="c", subcore_axis_name="s")

    @functools.partial(
        pl.kernel,
        mesh=mesh,
        out_type=jax.ShapeDtypeStruct((M, C), jnp.float32),
        scratch_types=[
            pltpu.VMEM((RSBUF,), jnp.int32),
            pltpu.VMEM((BLK,), jnp.int32),
            pltpu.VMEM((R, G, C), jnp.float32),
            pltpu.VMEM((W * C,), jnp.float32),
            pltpu.SemaphoreType.DMA((R,)),
            pltpu.SemaphoreType.DMA((W,)),
        ],
    )
    def pool(feat_hbm, idx_hbm, rs_hbm, out_hbm,
             rs_v, blk_v, g_v, row_v, gsem, wsem):
        wid = lax.axis_index("s") * NCORES + lax.axis_index("c")
        m0 = pl.multiple_of(wid * MPW, 8)
        mcount = jnp.minimum(MPW, M - m0)
        rs_astart = pl.multiple_of((m0 // 8) * 8, 8)
        rs_off = m0 - rs_astart
        pltpu.sync_copy(rs_hbm.at[pl.ds(rs_astart, RSBUF)], rs_v)

        def rs_at(i):
            return rs_v[pl.ds(rs_off + i, LAN)][0]

        e0 = rs_at(0)
        eN = rs_at(mcount)
        gstart = pl.multiple_of((e0 // 8) * 8, 8)
        nchunks = (eN - gstart + G - 1) // G
        nevents = nchunks + mcount

        zero = jnp.zeros((LAN,), jnp.float32)
        zeros_kc = (zero,) * KC

        def issue_gather(off, slot_ds):
            for h in range(G // LAN):
                idxv = blk_v[pl.ds(off + h * LAN, LAN)]
                pltpu.async_copy(feat_hbm.at[idxv],
                                 g_v.at[slot_ds, pl.ds(h * LAN, LAN)],
                                 gsem.at[slot_ds])

        @pl.when(nchunks > 0)
        def _():
            pltpu.sync_copy(idx_hbm.at[pl.ds(gstart, BLK)], blk_v)
            for i in range(R - 1):
                @pl.when(nchunks > i)
                def _(i=i):
                    issue_gather(i * G, i)
            pltpu.make_async_copy(feat_hbm.at[pl.ds(0, G)], g_v.at[0],
                                  gsem.at[0]).wait()

        def event(_, st):
            c, cur, m, seg_start, seg_end, bstart, accs = st
            p = lax.rem(c, R)
            cs = gstart + c * G
            cend = jnp.minimum(cs + G, eN)

            take = jnp.maximum(jnp.minimum(seg_end, cend) - cur, 0)
            lo = cur - cs

            def edge_body(e, a):
                return tuple(a[k] + g_v[p, e, pl.ds(k * LAN, LAN)]
                             for k in range(KC))

            accs = lax.fori_loop(lo, lo + take, edge_body, accs)
            cur = cur + take

            hit = jnp.logical_and(cur >= seg_end, m < mcount)
            adv = jnp.logical_and(jnp.logical_not(hit),
                                  jnp.logical_and(cur >= cend,
                                                  c + 1 < nchunks))

            @pl.when(hit)
            def _():
                ws2 = lax.rem(m, W)
                rbase = pl.multiple_of(ws2 * C, C)
                rslice = row_v.at[pl.ds(rbase, C)]

                @pl.when(m >= W)
                def _():
                    pltpu.make_async_copy(out_hbm.at[m0], rslice,
                                          wsem.at[ws2]).wait()

                cnt = seg_end - seg_start
                cntv = jnp.full((LAN,), cnt.astype(jnp.float32))
                recip = 1.0 / jnp.maximum(cntv, 1.0)
                for k in range(KC):
                    row_v[pl.ds(rbase + k * LAN, LAN)] = accs[k] * recip
                pltpu.async_copy(rslice, out_hbm.at[m0 + m], wsem.at[ws2])

            issue = jnp.logical_and(adv, c + R - 1 < nchunks)
            naddr = pl.multiple_of(gstart + (c + R - 1) * G, 8)
            restage = jnp.logical_and(issue, naddr + G > bstart + BLK)
            nbstart = jnp.where(restage, naddr, bstart)

            @pl.when(adv)
            def _():
                @pl.when(issue)
                def _():
                    @pl.when(restage)
                    def _():
                        pltpu.sync_copy(idx_hbm.at[pl.ds(naddr, BLK)], blk_v)

                    issue_gather(naddr - nbstart, lax.rem(c + R - 1, R))

                ws = lax.rem(c + 1, R)
                pltpu.make_async_copy(feat_hbm.at[pl.ds(0, G)],
                                      g_v.at[ws], gsem.at[ws]).wait()

            nm = jnp.where(hit, m + 1, m)
            seg_start = jnp.where(hit, seg_end, seg_start)
            seg_end = jnp.where(hit, rs_at(nm + 1), seg_end)
            accs = tuple(jnp.where(hit, zero, a) for a in accs)
            c = jnp.where(adv, c + 1, c)
            return (c, cur, nm, seg_start, seg_end, nbstart, accs)

        st0 = (jnp.int32(0), e0, jnp.int32(0), e0, rs_at(1),
               gstart, zeros_kc)
        lax.fori_loop(0, nevents, event, st0)

        for i in range(W):
            @pl.when(mcount > i)
            def _(i=i):
                ds = lax.rem(mcount - 1 - i, W)
                dbase = pl.multiple_of(ds * C, C)
                pltpu.make_async_copy(out_hbm.at[m0],
                                      row_v.at[pl.ds(dbase, C)],
                                      wsem.at[ds]).wait()

    return pool(in_features, idx32, rs32)
